# Initial kernel scaffold; baseline (speedup 1.0000x reference)
import functools

import jax
import jax.numpy as jnp
from jax import lax
from jax.experimental import pallas as pl
from jax.experimental.pallas import tpu as pltpu
from jax.experimental.pallas import tpu_sc as plsc

N = 10000
E = 320000
D = 128

NC = 2
NS = 16
NW = NC * NS
L = 16

EPW = E // NW
CHUNK = 80
NCHUNK = EPW // CHUNK
RPT = N // NS
ZROWS = 125

_mesh = plsc.VectorSubcoreMesh(core_axis_name="c", subcore_axis_name="s",
                               num_cores=NC, num_subcores=NS)
_f32 = jnp.float32


def _wid():
    return lax.axis_index("s") * NC + lax.axis_index("c")


def _zero_ref(ref, rows):
    zeros = jnp.zeros((L,), _f32)

    def row(r, _):
        for j in range(D // L):
            ref[r, pl.ds(j * L, L)] = zeros
        return 0

    lax.fori_loop(0, rows, row, 0, unroll=2)


def _zero_acc_slice(zbuf, acc):
    _zero_ref(zbuf, ZROWS)
    base = lax.axis_index("s") * RPT
    for q in range(RPT // ZROWS):
        pltpu.sync_copy(zbuf, acc.at[pl.ds(base + q * ZROWS, ZROWS)])


def _copy_out_slice(acc, out_hbm):
    c = lax.axis_index("c")
    base = lax.axis_index("s") * RPT
    pltpu.sync_copy(acc.at[pl.ds(base, RPT)], out_hbm.at[c, pl.ds(base, RPT)])


def _hist_body(src_hbm, dst_hbm, out_hbm, idx_s, idx_d, hist_s, hist_d):
    wid = _wid()
    pltpu.sync_copy(src_hbm.at[wid], idx_s)
    pltpu.sync_copy(dst_hbm.at[wid], idx_d)

    zeros = jnp.zeros((L,), _f32)

    def zb(i, _):
        hist_s[pl.ds(i * L, L)] = zeros
        hist_d[pl.ds(i * L, L)] = zeros
        return 0

    lax.fori_loop(0, N // L, zb, 0, unroll=4)

    ones = jnp.ones((L,), _f32)

    def hb(r, _):
        for j in range(CHUNK // L):
            vs = idx_s[r, pl.ds(j * L, L)]
            plsc.addupdate_scatter(hist_s, [vs], ones)
            vd = idx_d[r, pl.ds(j * L, L)]
            plsc.addupdate_scatter(hist_d, [vd], ones)
        return 0

    lax.fori_loop(0, NCHUNK, hb, 0)

    pltpu.sync_copy(hist_s, out_hbm.at[0, wid])
    pltpu.sync_copy(hist_d, out_hbm.at[1, wid])


_sc_hist = functools.partial(
    pl.kernel,
    out_type=jax.ShapeDtypeStruct((2, NW, N), _f32),
    mesh=_mesh,
    scratch_types=[
        pltpu.VMEM((NCHUNK, CHUNK), jnp.int32),
        pltpu.VMEM((NCHUNK, CHUNK), jnp.int32),
        pltpu.VMEM((N,), _f32),
        pltpu.VMEM((N,), _f32),
    ],
)(_hist_body)


def _gs_body(y_hbm, src_hbm, dst_hbm, out_hbm, idx_g, idx_sc, buf, zbuf, acc):
    wid = _wid()
    pltpu.sync_copy(src_hbm.at[wid], idx_g)
    pltpu.sync_copy(dst_hbm.at[wid], idx_sc)

    _zero_acc_slice(zbuf, acc)
    plsc.subcore_barrier()

    def chunk(t, _):
        pltpu.sync_copy(y_hbm.at[idx_g.at[t]], buf)
        pltpu.sync_copy(buf, acc.at[idx_sc.at[t]], add=True)
        return 0

    lax.fori_loop(0, NCHUNK, chunk, 0)

    plsc.subcore_barrier()
    _copy_out_slice(acc, out_hbm)


_sc_gs = functools.partial(
    pl.kernel,
    out_type=jax.ShapeDtypeStruct((NC, N, D), _f32),
    mesh=_mesh,
    scratch_types=[
        pltpu.VMEM((NCHUNK, CHUNK), jnp.int32),
        pltpu.VMEM((NCHUNK, CHUNK), jnp.int32),
        pltpu.VMEM((CHUNK, D), _f32),
        pltpu.VMEM((ZROWS, D), _f32),
        pltpu.VMEM_SHARED((N, D), _f32),
    ],
)(_gs_body)


def _diff_body(h_hbm, src_hbm, dst_hbm, out_hbm,
               idx_s, idx_d, buf_a, buf_b, buf_c, zbuf, acc):
    wid = _wid()
    pltpu.sync_copy(src_hbm.at[wid], idx_s)
    pltpu.sync_copy(dst_hbm.at[wid], idx_d)

    _zero_acc_slice(zbuf, acc)
    plsc.subcore_barrier()

    def chunk(t, _):
        pltpu.sync_copy(h_hbm.at[idx_s.at[t]], buf_a)
        pltpu.sync_copy(h_hbm.at[idx_d.at[t]], buf_b)

        def row(r, _):
            for j in range(D // L):
                a = buf_a[r, pl.ds(j * L, L)]
                b = buf_b[r, pl.ds(j * L, L)]
                d = a - b
                buf_c[r, pl.ds(j * L, L)] = d * d
            return 0

        lax.fori_loop(0, CHUNK, row, 0, unroll=2)
        pltpu.sync_copy(buf_c, acc.at[idx_s.at[t]], add=True)
        return 0

    lax.fori_loop(0, NCHUNK, chunk, 0)

    plsc.subcore_barrier()
    _copy_out_slice(acc, out_hbm)


_sc_diff = functools.partial(
    pl.kernel,
    out_type=jax.ShapeDtypeStruct((NC, N, D), _f32),
    mesh=_mesh,
    scratch_types=[
        pltpu.VMEM((NCHUNK, CHUNK), jnp.int32),
        pltpu.VMEM((NCHUNK, CHUNK), jnp.int32),
        pltpu.VMEM((CHUNK, D), _f32),
        pltpu.VMEM((CHUNK, D), _f32),
        pltpu.VMEM((CHUNK, D), _f32),
        pltpu.VMEM((ZROWS, D), _f32),
        pltpu.VMEM_SHARED((N, D), _f32),
    ],
)(_diff_body)


_TCB = 2000


def _tc_prep_body(x_ref, w_ref, hist_ref, y_ref):
    deg = 1.0 + jnp.sum(hist_ref[...], axis=0)
    dinv = lax.rsqrt(deg)
    xw = jnp.dot(x_ref[...], w_ref[...], preferred_element_type=_f32)
    y_ref[...] = xw * dinv[:, None]


def _tc_prep(X, W, hist_dst):
    return pl.pallas_call(
        _tc_prep_body,
        grid=(N // _TCB,),
        in_specs=[
            pl.BlockSpec((_TCB, D), lambda i: (i, 0)),
            pl.BlockSpec((D, D), lambda i: (0, 0)),
            pl.BlockSpec((NW, _TCB), lambda i: (0, i)),
        ],
        out_specs=pl.BlockSpec((_TCB, D), lambda i: (i, 0)),
        out_shape=jax.ShapeDtypeStruct((N, D), _f32),
    )(X, W, hist_dst)


def _tc_combine_body(s1a_ref, s1b_ref, y_ref, hist_ref, b_ref, h_ref):
    deg = 1.0 + jnp.sum(hist_ref[...], axis=0)
    dinv = lax.rsqrt(deg)
    h = dinv[:, None] * (s1a_ref[...] + s1b_ref[...] + y_ref[...]) + b_ref[...]
    h_ref[...] = jnp.maximum(h, 0.0)


def _tc_combine(s1a, s1b, Y, hist_dst, b2d):
    return pl.pallas_call(
        _tc_combine_body,
        grid=(N // _TCB,),
        in_specs=[
            pl.BlockSpec((_TCB, D), lambda i: (i, 0)),
            pl.BlockSpec((_TCB, D), lambda i: (i, 0)),
            pl.BlockSpec((_TCB, D), lambda i: (i, 0)),
            pl.BlockSpec((NW, _TCB), lambda i: (0, i)),
            pl.BlockSpec((1, D), lambda i: (0, 0)),
        ],
        out_specs=pl.BlockSpec((_TCB, D), lambda i: (i, 0)),
        out_shape=jax.ShapeDtypeStruct((N, D), _f32),
    )(s1a, s1b, Y, hist_dst, b2d)


def _tc_final_body(s2a_ref, s2b_ref, hist_ref, g_ref):
    cnt = jnp.sum(hist_ref[...], axis=0)
    mean = (s2a_ref[...] + s2b_ref[...]) / jnp.maximum(cnt, 1.0)[:, None]
    g_ref[...] = jnp.tanh(mean)


def _tc_final(s2a, s2b, hist_src):
    return pl.pallas_call(
        _tc_final_body,
        grid=(N // _TCB,),
        in_specs=[
            pl.BlockSpec((_TCB, D), lambda i: (i, 0)),
            pl.BlockSpec((_TCB, D), lambda i: (i, 0)),
            pl.BlockSpec((NW, _TCB), lambda i: (0, i)),
        ],
        out_specs=pl.BlockSpec((_TCB, D), lambda i: (i, 0)),
        out_shape=jax.ShapeDtypeStruct((N, D), _f32),
    )(s2a, s2b, hist_src)


@jax.jit
def kernel(X, edge_index, W, b):
    src = edge_index[0]
    dst = edge_index[1]
    srcR = src.reshape(NW, NCHUNK, CHUNK)
    dstR = dst.reshape(NW, NCHUNK, CHUNK)

    hists = _sc_hist(srcR, dstR)
    hist_src = hists[0]
    hist_dst = hists[1]

    Y = _tc_prep(X, W, hist_dst)
    S1 = _sc_gs(Y, srcR, dstR)
    H = _tc_combine(S1[0], S1[1], Y, hist_dst, b.reshape(1, D))
    S2 = _sc_diff(H, srcR, dstR)
    return _tc_final(S2[0], S2[1], hist_src)


# speedup vs baseline: 8.8128x; 8.8128x over previous
import functools

import jax
import jax.numpy as jnp
from jax import lax
from jax.experimental import pallas as pl
from jax.experimental.pallas import tpu as pltpu
from jax.experimental.pallas import tpu_sc as plsc

N = 10000
E = 320000
D = 128

NC = 2
NS = 16
NW = NC * NS
L = 16

EPW = E // NW
CHUNK = 80
NCHUNK = EPW // CHUNK
RPT = N // NS

_mesh = plsc.VectorSubcoreMesh(core_axis_name="c", subcore_axis_name="s",
                               num_cores=NC, num_subcores=NS)
_f32 = jnp.float32
_sc_params = pltpu.CompilerParams(needs_layout_passes=False,
                                  use_tc_tiling_on_sc=False)


def _wid():
    return lax.axis_index("s") * NC + lax.axis_index("c")


def _zero_ref(ref, rows):
    zeros = jnp.zeros((L,), _f32)

    def row(r, _):
        for j in range(D // L):
            ref[r, pl.ds(j * L, L)] = zeros
        return 0

    lax.fori_loop(0, rows, row, 0, unroll=2)


def _zero_acc_slice(buf, acc):
    _zero_ref(buf, CHUNK)
    base = lax.axis_index("s") * RPT
    for q in range(RPT // CHUNK):
        pltpu.sync_copy(buf, acc.at[pl.ds(base + q * CHUNK, CHUNK)])
    rem = RPT % CHUNK
    pltpu.sync_copy(buf.at[pl.ds(0, rem)],
                    acc.at[pl.ds(base + RPT - rem, rem)])


def _copy_out_slice(acc, out_hbm):
    c = lax.axis_index("c")
    base = lax.axis_index("s") * RPT
    pltpu.sync_copy(acc.at[pl.ds(base, RPT)], out_hbm.at[c, pl.ds(base, RPT)])


def _hist_body(src_hbm, dst_hbm, out_hbm, idx_s, idx_d, hist_s, hist_d):
    wid = _wid()
    pltpu.sync_copy(src_hbm.at[wid], idx_s)
    pltpu.sync_copy(dst_hbm.at[wid], idx_d)

    zeros = jnp.zeros((L,), _f32)

    def zb(i, _):
        hist_s[pl.ds(i * L, L)] = zeros
        hist_d[pl.ds(i * L, L)] = zeros
        return 0

    lax.fori_loop(0, N // L, zb, 0, unroll=4)

    ones = jnp.ones((L,), _f32)

    def hb(r, _):
        for j in range(CHUNK // L):
            vs = idx_s[r, pl.ds(j * L, L)]
            plsc.addupdate_scatter(hist_s, [vs], ones)
            vd = idx_d[r, pl.ds(j * L, L)]
            plsc.addupdate_scatter(hist_d, [vd], ones)
        return 0

    lax.fori_loop(0, NCHUNK, hb, 0)

    pltpu.sync_copy(hist_s, out_hbm.at[0, wid])
    pltpu.sync_copy(hist_d, out_hbm.at[1, wid])


_sc_hist = functools.partial(
    pl.kernel,
    out_type=jax.ShapeDtypeStruct((2, NW, N), _f32),
    mesh=_mesh,
    compiler_params=_sc_params,
    scratch_types=[
        pltpu.VMEM((NCHUNK, CHUNK), jnp.int32),
        pltpu.VMEM((NCHUNK, CHUNK), jnp.int32),
        pltpu.VMEM((N,), _f32),
        pltpu.VMEM((N,), _f32),
    ],
)(_hist_body)


def _gs_body(y_hbm, src_hbm, dst_hbm, out_hbm, idx_g, idx_sc, buf, acc):
    wid = _wid()
    pltpu.sync_copy(src_hbm.at[wid], idx_g)
    pltpu.sync_copy(dst_hbm.at[wid], idx_sc)

    _zero_acc_slice(buf, acc)
    plsc.subcore_barrier()

    def chunk(t, _):
        pltpu.sync_copy(y_hbm.at[idx_g.at[t]], buf)
        pltpu.sync_copy(buf, acc.at[idx_sc.at[t]], add=True)
        return 0

    lax.fori_loop(0, NCHUNK, chunk, 0)

    plsc.subcore_barrier()
    _copy_out_slice(acc, out_hbm)


_sc_gs = functools.partial(
    pl.kernel,
    out_type=jax.ShapeDtypeStruct((NC, N, D), _f32),
    mesh=_mesh,
    compiler_params=_sc_params,
    scratch_types=[
        pltpu.VMEM((NCHUNK, CHUNK), jnp.int32),
        pltpu.VMEM((NCHUNK, CHUNK), jnp.int32),
        pltpu.VMEM((CHUNK, D), _f32),
        pltpu.VMEM_SHARED((N, D), _f32),
    ],
)(_gs_body)


def _diff_body(h_hbm, src_hbm, dst_hbm, out_hbm,
               idx_s, idx_d, buf_a, buf_b, buf_c, acc):
    wid = _wid()
    pltpu.sync_copy(src_hbm.at[wid], idx_s)
    pltpu.sync_copy(dst_hbm.at[wid], idx_d)

    _zero_acc_slice(buf_c, acc)
    plsc.subcore_barrier()

    def chunk(t, _):
        pltpu.sync_copy(h_hbm.at[idx_s.at[t]], buf_a)
        pltpu.sync_copy(h_hbm.at[idx_d.at[t]], buf_b)

        def row(r, _):
            for j in range(D // L):
                a = buf_a[r, pl.ds(j * L, L)]
                b = buf_b[r, pl.ds(j * L, L)]
                d = a - b
                buf_c[r, pl.ds(j * L, L)] = d * d
            return 0

        lax.fori_loop(0, CHUNK, row, 0, unroll=2)
        pltpu.sync_copy(buf_c, acc.at[idx_s.at[t]], add=True)
        return 0

    lax.fori_loop(0, NCHUNK, chunk, 0)

    plsc.subcore_barrier()
    _copy_out_slice(acc, out_hbm)


_sc_diff = functools.partial(
    pl.kernel,
    out_type=jax.ShapeDtypeStruct((NC, N, D), _f32),
    mesh=_mesh,
    compiler_params=_sc_params,
    scratch_types=[
        pltpu.VMEM((NCHUNK, CHUNK), jnp.int32),
        pltpu.VMEM((NCHUNK, CHUNK), jnp.int32),
        pltpu.VMEM((CHUNK, D), _f32),
        pltpu.VMEM((CHUNK, D), _f32),
        pltpu.VMEM((CHUNK, D), _f32),
        pltpu.VMEM_SHARED((N, D), _f32),
    ],
)(_diff_body)


_TCB = 2000


def _tc_prep_body(x_ref, w_ref, hist_ref, y_ref):
    deg = 1.0 + jnp.sum(hist_ref[...], axis=1)
    dinv = lax.rsqrt(deg)
    xw = jnp.dot(x_ref[...], w_ref[...], preferred_element_type=_f32)
    y_ref[...] = xw * dinv[:, None]


def _tc_prep(X, W, hist_dst):
    return pl.pallas_call(
        _tc_prep_body,
        grid=(N // _TCB,),
        in_specs=[
            pl.BlockSpec((_TCB, D), lambda i: (i, 0)),
            pl.BlockSpec((D, D), lambda i: (0, 0)),
            pl.BlockSpec((_TCB, NW), lambda i: (i, 0)),
        ],
        out_specs=pl.BlockSpec((_TCB, D), lambda i: (i, 0)),
        out_shape=jax.ShapeDtypeStruct((N, D), _f32),
    )(X, W, hist_dst)


def _tc_combine_body(s1a_ref, s1b_ref, y_ref, hist_ref, b_ref, h_ref):
    deg = 1.0 + jnp.sum(hist_ref[...], axis=1)
    dinv = lax.rsqrt(deg)
    h = dinv[:, None] * (s1a_ref[...] + s1b_ref[...] + y_ref[...]) + b_ref[...]
    h_ref[...] = jnp.maximum(h, 0.0)


def _tc_combine(s1a, s1b, Y, hist_dst, b2d):
    return pl.pallas_call(
        _tc_combine_body,
        grid=(N // _TCB,),
        in_specs=[
            pl.BlockSpec((_TCB, D), lambda i: (i, 0)),
            pl.BlockSpec((_TCB, D), lambda i: (i, 0)),
            pl.BlockSpec((_TCB, D), lambda i: (i, 0)),
            pl.BlockSpec((_TCB, NW), lambda i: (i, 0)),
            pl.BlockSpec((1, D), lambda i: (0, 0)),
        ],
        out_specs=pl.BlockSpec((_TCB, D), lambda i: (i, 0)),
        out_shape=jax.ShapeDtypeStruct((N, D), _f32),
    )(s1a, s1b, Y, hist_dst, b2d)


def _tc_final_body(s2a_ref, s2b_ref, hist_ref, g_ref):
    cnt = jnp.sum(hist_ref[...], axis=1)
    mean = (s2a_ref[...] + s2b_ref[...]) / jnp.maximum(cnt, 1.0)[:, None]
    g_ref[...] = jnp.tanh(mean)


def _tc_final(s2a, s2b, hist_src):
    return pl.pallas_call(
        _tc_final_body,
        grid=(N // _TCB,),
        in_specs=[
            pl.BlockSpec((_TCB, D), lambda i: (i, 0)),
            pl.BlockSpec((_TCB, D), lambda i: (i, 0)),
            pl.BlockSpec((_TCB, NW), lambda i: (i, 0)),
        ],
        out_specs=pl.BlockSpec((_TCB, D), lambda i: (i, 0)),
        out_shape=jax.ShapeDtypeStruct((N, D), _f32),
    )(s2a, s2b, hist_src)


@jax.jit
def kernel(X, edge_index, W, b):
    src = edge_index[0]
    dst = edge_index[1]
    srcR = src.reshape(NW, NCHUNK, CHUNK)
    dstR = dst.reshape(NW, NCHUNK, CHUNK)

    hists = _sc_hist(srcR, dstR)
    hist_src = hists[0].T
    hist_dst = hists[1].T

    Y = _tc_prep(X, W, hist_dst)
    S1 = _sc_gs(Y, srcR, dstR)
    H = _tc_combine(S1[0], S1[1], Y, hist_dst, b.reshape(1, D))
    S2 = _sc_diff(H, srcR, dstR)
    return _tc_final(S2[0], S2[1], hist_src)


# speedup vs baseline: 12.6637x; 1.4370x over previous
import functools

import jax
import jax.numpy as jnp
from jax import lax
from jax.experimental import pallas as pl
from jax.experimental.pallas import tpu as pltpu
from jax.experimental.pallas import tpu_sc as plsc

N = 10000
E = 320000
D = 128

NC = 2
NS = 16
NW = NC * NS
L = 16

EPW = E // NW
CHUNK = 80
NCHUNK = EPW // CHUNK
DCHUNK = 40
DNCHUNK = EPW // DCHUNK
RPT = N // NS

_mesh = plsc.VectorSubcoreMesh(core_axis_name="c", subcore_axis_name="s",
                               num_cores=NC, num_subcores=NS)
_f32 = jnp.float32
_sc_params = pltpu.CompilerParams(needs_layout_passes=False,
                                  use_tc_tiling_on_sc=False)


def _wid():
    return lax.axis_index("s") * NC + lax.axis_index("c")


def _zero_ref(ref, rows):
    zeros = jnp.zeros((L,), _f32)

    def row(r, _):
        for j in range(D // L):
            ref[r, pl.ds(j * L, L)] = zeros
        return 0

    lax.fori_loop(0, rows, row, 0, unroll=2)


def _zero_acc_slice(buf, rows, acc):
    _zero_ref(buf, rows)
    base = lax.axis_index("s") * RPT
    for q in range(RPT // rows):
        pltpu.sync_copy(buf, acc.at[pl.ds(base + q * rows, rows)])
    rem = RPT % rows
    if rem:
        pltpu.sync_copy(buf.at[pl.ds(0, rem)],
                        acc.at[pl.ds(base + RPT - rem, rem)])


def _copy_out_slice(acc, out_hbm):
    c = lax.axis_index("c")
    base = lax.axis_index("s") * RPT
    pltpu.sync_copy(acc.at[pl.ds(base, RPT)], out_hbm.at[c, pl.ds(base, RPT)])


def _hist_body(src_hbm, dst_hbm, out_hbm, idx_s, idx_d, hist_s, hist_d):
    wid = _wid()
    pltpu.sync_copy(src_hbm.at[wid], idx_s)
    pltpu.sync_copy(dst_hbm.at[wid], idx_d)

    zeros = jnp.zeros((L,), _f32)

    def zb(i, _):
        hist_s[pl.ds(i * L, L)] = zeros
        hist_d[pl.ds(i * L, L)] = zeros
        return 0

    lax.fori_loop(0, N // L, zb, 0, unroll=4)

    ones = jnp.ones((L,), _f32)

    def hb(r, _):
        for j in range(CHUNK // L):
            vs = idx_s[r, pl.ds(j * L, L)]
            plsc.addupdate_scatter(hist_s, [vs], ones)
            vd = idx_d[r, pl.ds(j * L, L)]
            plsc.addupdate_scatter(hist_d, [vd], ones)
        return 0

    lax.fori_loop(0, NCHUNK, hb, 0)

    pltpu.sync_copy(hist_s, out_hbm.at[0, wid])
    pltpu.sync_copy(hist_d, out_hbm.at[1, wid])


_sc_hist = functools.partial(
    pl.kernel,
    out_type=jax.ShapeDtypeStruct((2, NW, N), _f32),
    mesh=_mesh,
    compiler_params=_sc_params,
    scratch_types=[
        pltpu.VMEM((NCHUNK, CHUNK), jnp.int32),
        pltpu.VMEM((NCHUNK, CHUNK), jnp.int32),
        pltpu.VMEM((N,), _f32),
        pltpu.VMEM((N,), _f32),
    ],
)(_hist_body)


def _gs_body(y_hbm, src_hbm, dst_hbm, out_hbm,
             idx_g, idx_sc, buf0, buf1, acc, sem0, sem1):
    wid = _wid()
    pltpu.sync_copy(src_hbm.at[wid], idx_g)
    pltpu.sync_copy(dst_hbm.at[wid], idx_sc)

    _zero_acc_slice(buf0, CHUNK, acc)
    plsc.subcore_barrier()

    bufs = (buf0, buf1)
    sems = (sem0, sem1)

    def fire(t, p):
        pltpu.async_copy(y_hbm.at[idx_g.at[t]], bufs[p], sems[p])

    def wait(p):
        pltpu.make_async_copy(y_hbm.at[idx_g.at[0]], bufs[p], sems[p]).wait()

    def scat(t, p):
        pltpu.sync_copy(bufs[p], acc.at[idx_sc.at[t]], add=True)

    fire(0, 0)

    def pair(i, _):
        t0 = 2 * i
        fire(t0 + 1, 1)
        wait(0)
        scat(t0, 0)
        fire(t0 + 2, 0)
        wait(1)
        scat(t0 + 1, 1)
        return 0

    lax.fori_loop(0, NCHUNK // 2, pair, 0)
    wait(0)
    scat(NCHUNK - 1, 0)

    plsc.subcore_barrier()
    _copy_out_slice(acc, out_hbm)


_sc_gs = functools.partial(
    pl.kernel,
    out_type=jax.ShapeDtypeStruct((NC, N, D), _f32),
    mesh=_mesh,
    compiler_params=_sc_params,
    scratch_types=[
        pltpu.VMEM((NCHUNK, CHUNK), jnp.int32),
        pltpu.VMEM((NCHUNK, CHUNK), jnp.int32),
        pltpu.VMEM((CHUNK, D), _f32),
        pltpu.VMEM((CHUNK, D), _f32),
        pltpu.VMEM_SHARED((N, D), _f32),
        pltpu.SemaphoreType.DMA,
        pltpu.SemaphoreType.DMA,
    ],
)(_gs_body)


def _diff_body(h_hbm, src_hbm, dst_hbm, out_hbm,
               idx_s, idx_d, buf_a0, buf_b0, buf_a1, buf_b1, buf_c, acc,
               sem0, sem1):
    wid = _wid()
    pltpu.sync_copy(src_hbm.at[wid], idx_s)
    pltpu.sync_copy(dst_hbm.at[wid], idx_d)

    _zero_acc_slice(buf_c, DCHUNK, acc)
    plsc.subcore_barrier()

    bufs_a = (buf_a0, buf_a1)
    bufs_b = (buf_b0, buf_b1)
    sems = (sem0, sem1)

    def fire(t, p):
        pltpu.async_copy(h_hbm.at[idx_s.at[t]], bufs_a[p], sems[p])
        pltpu.async_copy(h_hbm.at[idx_d.at[t]], bufs_b[p], sems[p])

    def wait(p):
        pltpu.make_async_copy(h_hbm.at[idx_s.at[0]], bufs_a[p], sems[p]).wait()
        pltpu.make_async_copy(h_hbm.at[idx_s.at[0]], bufs_b[p], sems[p]).wait()

    def work(t, p):
        a_ref = bufs_a[p]
        b_ref = bufs_b[p]

        def row(r, _):
            for j in range(D // L):
                a = a_ref[r, pl.ds(j * L, L)]
                b = b_ref[r, pl.ds(j * L, L)]
                d = a - b
                buf_c[r, pl.ds(j * L, L)] = d * d
            return 0

        lax.fori_loop(0, DCHUNK, row, 0, unroll=4)
        pltpu.sync_copy(buf_c, acc.at[idx_s.at[t]], add=True)

    fire(0, 0)

    def pair(i, _):
        t0 = 2 * i
        fire(t0 + 1, 1)
        wait(0)
        work(t0, 0)
        fire(t0 + 2, 0)
        wait(1)
        work(t0 + 1, 1)
        return 0

    lax.fori_loop(0, DNCHUNK // 2 - 1, pair, 0)
    t0 = DNCHUNK - 2
    fire(t0 + 1, 1)
    wait(0)
    work(t0, 0)
    wait(1)
    work(t0 + 1, 1)

    plsc.subcore_barrier()
    _copy_out_slice(acc, out_hbm)


_sc_diff = functools.partial(
    pl.kernel,
    out_type=jax.ShapeDtypeStruct((NC, N, D), _f32),
    mesh=_mesh,
    compiler_params=_sc_params,
    scratch_types=[
        pltpu.VMEM((DNCHUNK, DCHUNK), jnp.int32),
        pltpu.VMEM((DNCHUNK, DCHUNK), jnp.int32),
        pltpu.VMEM((DCHUNK, D), _f32),
        pltpu.VMEM((DCHUNK, D), _f32),
        pltpu.VMEM((DCHUNK, D), _f32),
        pltpu.VMEM((DCHUNK, D), _f32),
        pltpu.VMEM((DCHUNK, D), _f32),
        pltpu.VMEM_SHARED((N, D), _f32),
        pltpu.SemaphoreType.DMA,
        pltpu.SemaphoreType.DMA,
    ],
)(_diff_body)


_TCB = 2000


def _tc_prep_body(x_ref, w_ref, hist_ref, y_ref):
    deg = 1.0 + jnp.sum(hist_ref[...], axis=1)
    dinv = lax.rsqrt(deg)
    xw = jnp.dot(x_ref[...], w_ref[...], preferred_element_type=_f32)
    y_ref[...] = xw * dinv[:, None]


def _tc_prep(X, W, hist_dst):
    return pl.pallas_call(
        _tc_prep_body,
        grid=(N // _TCB,),
        in_specs=[
            pl.BlockSpec((_TCB, D), lambda i: (i, 0)),
            pl.BlockSpec((D, D), lambda i: (0, 0)),
            pl.BlockSpec((_TCB, NW), lambda i: (i, 0)),
        ],
        out_specs=pl.BlockSpec((_TCB, D), lambda i: (i, 0)),
        out_shape=jax.ShapeDtypeStruct((N, D), _f32),
    )(X, W, hist_dst)


def _tc_combine_body(s1a_ref, s1b_ref, y_ref, hist_ref, b_ref, h_ref):
    deg = 1.0 + jnp.sum(hist_ref[...], axis=1)
    dinv = lax.rsqrt(deg)
    h = dinv[:, None] * (s1a_ref[...] + s1b_ref[...] + y_ref[...]) + b_ref[...]
    h_ref[...] = jnp.maximum(h, 0.0)


def _tc_combine(s1a, s1b, Y, hist_dst, b2d):
    return pl.pallas_call(
        _tc_combine_body,
        grid=(N // _TCB,),
        in_specs=[
            pl.BlockSpec((_TCB, D), lambda i: (i, 0)),
            pl.BlockSpec((_TCB, D), lambda i: (i, 0)),
            pl.BlockSpec((_TCB, D), lambda i: (i, 0)),
            pl.BlockSpec((_TCB, NW), lambda i: (i, 0)),
            pl.BlockSpec((1, D), lambda i: (0, 0)),
        ],
        out_specs=pl.BlockSpec((_TCB, D), lambda i: (i, 0)),
        out_shape=jax.ShapeDtypeStruct((N, D), _f32),
    )(s1a, s1b, Y, hist_dst, b2d)


def _tc_final_body(s2a_ref, s2b_ref, hist_ref, g_ref):
    cnt = jnp.sum(hist_ref[...], axis=1)
    mean = (s2a_ref[...] + s2b_ref[...]) / jnp.maximum(cnt, 1.0)[:, None]
    g_ref[...] = jnp.tanh(mean)


def _tc_final(s2a, s2b, hist_src):
    return pl.pallas_call(
        _tc_final_body,
        grid=(N // _TCB,),
        in_specs=[
            pl.BlockSpec((_TCB, D), lambda i: (i, 0)),
            pl.BlockSpec((_TCB, D), lambda i: (i, 0)),
            pl.BlockSpec((_TCB, NW), lambda i: (i, 0)),
        ],
        out_specs=pl.BlockSpec((_TCB, D), lambda i: (i, 0)),
        out_shape=jax.ShapeDtypeStruct((N, D), _f32),
    )(s2a, s2b, hist_src)


@jax.jit
def kernel(X, edge_index, W, b):
    src = edge_index[0]
    dst = edge_index[1]
    srcR = src.reshape(NW, NCHUNK, CHUNK)
    dstR = dst.reshape(NW, NCHUNK, CHUNK)
    srcD = src.reshape(NW, DNCHUNK, DCHUNK)
    dstD = dst.reshape(NW, DNCHUNK, DCHUNK)

    hists = _sc_hist(srcR, dstR)
    hist_src = hists[0].T
    hist_dst = hists[1].T

    Y = _tc_prep(X, W, hist_dst)
    S1 = _sc_gs(Y, srcR, dstR)
    H = _tc_combine(S1[0], S1[1], Y, hist_dst, b.reshape(1, D))
    S2 = _sc_diff(H, srcD, dstD)
    return _tc_final(S2[0], S2[1], hist_src)


# speedup vs baseline: 27.4881x; 2.1706x over previous
import functools

import jax
import jax.numpy as jnp
from jax import lax
from jax.experimental import pallas as pl
from jax.experimental.pallas import tpu as pltpu
from jax.experimental.pallas import tpu_sc as plsc

N = 10000
E = 320000
D = 128

NC = 2
NS = 16
NW = NC * NS
L = 16

HCH = 80
HNCH = (E // NW) // HCH

C = 125
NCH_GS = (E // NW) // C
NCH_AB = (E // NS) // C
RPT = N // NS

_mesh = plsc.VectorSubcoreMesh(core_axis_name="c", subcore_axis_name="s",
                               num_cores=NC, num_subcores=NS)
_f32 = jnp.float32
_sc_params = pltpu.CompilerParams(needs_layout_passes=False,
                                  use_tc_tiling_on_sc=False)


def _zero_ref(ref, rows):
    zeros = jnp.zeros((L,), _f32)

    def row(r, _):
        for j in range(D // L):
            ref[r, pl.ds(j * L, L)] = zeros
        return 0

    lax.fori_loop(0, rows, row, 0, unroll=2)


def _zero_acc_slice(buf, rows, acc):
    _zero_ref(buf, rows)
    base = lax.axis_index("s") * RPT
    for q in range(RPT // rows):
        pltpu.sync_copy(buf, acc.at[pl.ds(base + q * rows, rows)])
    rem = RPT % rows
    if rem:
        pltpu.sync_copy(buf.at[pl.ds(0, rem)],
                        acc.at[pl.ds(base + RPT - rem, rem)])


def _copy_out_slice(acc, out_hbm):
    c = lax.axis_index("c")
    base = lax.axis_index("s") * RPT
    pltpu.sync_copy(acc.at[pl.ds(base, RPT)], out_hbm.at[c, pl.ds(base, RPT)])


def _hist_body(src_hbm, dst_hbm, out_hbm, idx_s, idx_d, hist_s, hist_d):
    wid = lax.axis_index("s") * NC + lax.axis_index("c")
    pltpu.sync_copy(src_hbm.at[wid], idx_s)
    pltpu.sync_copy(dst_hbm.at[wid], idx_d)

    zeros = jnp.zeros((L,), _f32)

    def zb(i, _):
        hist_s[pl.ds(i * L, L)] = zeros
        hist_d[pl.ds(i * L, L)] = zeros
        return 0

    lax.fori_loop(0, N // L, zb, 0, unroll=4)

    ones = jnp.ones((L,), _f32)

    def hb(r, _):
        for j in range(HCH // L):
            vs = idx_s[r, pl.ds(j * L, L)]
            plsc.addupdate_scatter(hist_s, [vs], ones)
            vd = idx_d[r, pl.ds(j * L, L)]
            plsc.addupdate_scatter(hist_d, [vd], ones)
        return 0

    lax.fori_loop(0, HNCH, hb, 0)

    pltpu.sync_copy(hist_s, out_hbm.at[0, wid])
    pltpu.sync_copy(hist_d, out_hbm.at[1, wid])


_sc_hist = functools.partial(
    pl.kernel,
    out_type=jax.ShapeDtypeStruct((2, NW, N), _f32),
    mesh=_mesh,
    compiler_params=_sc_params,
    scratch_types=[
        pltpu.VMEM((HNCH, HCH), jnp.int32),
        pltpu.VMEM((HNCH, HCH), jnp.int32),
        pltpu.VMEM((N,), _f32),
        pltpu.VMEM((N,), _f32),
    ],
)(_hist_body)


def _gscat_pipeline(table_hbm, gidx_hbm, sidx_hbm, acc, gbufs, gsems,
                    ibuf_g, ibuf_s, isems, c, sid, nch):
    def fire_i(t, s):
        pltpu.async_copy(gidx_hbm.at[c, sid, t], ibuf_g.at[s], isems[s])
        pltpu.async_copy(sidx_hbm.at[c, sid, t], ibuf_s.at[s], isems[s])

    def wait_i(s):
        pltpu.make_async_copy(gidx_hbm.at[c, sid, 0], ibuf_g.at[s],
                              isems[s]).wait()
        pltpu.make_async_copy(gidx_hbm.at[c, sid, 0], ibuf_s.at[s],
                              isems[s]).wait()

    def fire_g(s, b):
        pltpu.async_copy(table_hbm.at[ibuf_g.at[s]], gbufs[b], gsems[b])

    def wait_g(b):
        pltpu.make_async_copy(table_hbm.at[ibuf_g.at[0]], gbufs[b],
                              gsems[b]).wait()

    def scat(s, b):
        pltpu.sync_copy(gbufs[b], acc.at[ibuf_s.at[s]], add=True)

    for s in range(4):
        fire_i(s, s)
    wait_i(0)
    fire_g(0, 0)
    wait_i(1)
    fire_g(1, 1)

    main12 = (nch - 4) // 12

    def block(i, _):
        tb = 12 * i
        for u in range(12):
            wait_i((u + 2) % 4)
            fire_g((u + 2) % 4, (u + 2) % 3)
            wait_g(u % 3)
            scat(u % 4, u % 3)
            fire_i(tb + u + 4, u % 4)
        return 0

    lax.fori_loop(0, main12, block, 0)

    for t in range(12 * main12, nch):
        if t + 2 < nch:
            wait_i((t + 2) % 4)
            fire_g((t + 2) % 4, (t + 2) % 3)
        wait_g(t % 3)
        scat(t % 4, t % 3)
        if t + 4 < nch:
            fire_i(t + 4, t % 4)


def _make_gscat(nch):
    def body(table_hbm, gidx_hbm, sidx_hbm, out_hbm,
             gbuf0, gbuf1, gbuf2, ibuf_g, ibuf_s, acc,
             gsem0, gsem1, gsem2, isem0, isem1, isem2, isem3):
        c = lax.axis_index("c")
        sid = lax.axis_index("s")
        _zero_acc_slice(gbuf0, C, acc)
        plsc.subcore_barrier()
        _gscat_pipeline(table_hbm, gidx_hbm, sidx_hbm, acc,
                        (gbuf0, gbuf1, gbuf2),
                        (gsem0, gsem1, gsem2),
                        ibuf_g, ibuf_s,
                        (isem0, isem1, isem2, isem3),
                        c, sid, nch)
        plsc.subcore_barrier()
        _copy_out_slice(acc, out_hbm)

    return functools.partial(
        pl.kernel,
        out_type=jax.ShapeDtypeStruct((NC, N, D), _f32),
        mesh=_mesh,
        compiler_params=_sc_params,
        scratch_types=[
            pltpu.VMEM((C, D), _f32),
            pltpu.VMEM((C, D), _f32),
            pltpu.VMEM((C, D), _f32),
            pltpu.VMEM((4, C), jnp.int32),
            pltpu.VMEM((4, C), jnp.int32),
            pltpu.VMEM_SHARED((N, D), _f32),
            pltpu.SemaphoreType.DMA,
            pltpu.SemaphoreType.DMA,
            pltpu.SemaphoreType.DMA,
            pltpu.SemaphoreType.DMA,
            pltpu.SemaphoreType.DMA,
            pltpu.SemaphoreType.DMA,
            pltpu.SemaphoreType.DMA,
        ],
    )(body)


_sc_gs = _make_gscat(NCH_GS)
_sc_ab = _make_gscat(NCH_AB)


_TCB = 2000


def _tc_prep_body(x_ref, w_ref, hist_ref, y_ref):
    deg = 1.0 + jnp.sum(hist_ref[...], axis=1)
    dinv = lax.rsqrt(deg)
    xw = jnp.dot(x_ref[...], w_ref[...], preferred_element_type=_f32)
    y_ref[...] = xw * dinv[:, None]


def _tc_prep(X, W, hist_dst):
    return pl.pallas_call(
        _tc_prep_body,
        grid=(N // _TCB,),
        in_specs=[
            pl.BlockSpec((_TCB, D), lambda i: (i, 0)),
            pl.BlockSpec((D, D), lambda i: (0, 0)),
            pl.BlockSpec((_TCB, NW), lambda i: (i, 0)),
        ],
        out_specs=pl.BlockSpec((_TCB, D), lambda i: (i, 0)),
        out_shape=jax.ShapeDtypeStruct((N, D), _f32),
    )(X, W, hist_dst)


def _tc_combine_body(s1a_ref, s1b_ref, y_ref, hist_ref, b_ref, h_ref):
    deg = 1.0 + jnp.sum(hist_ref[...], axis=1)
    dinv = lax.rsqrt(deg)
    h = dinv[:, None] * (s1a_ref[...] + s1b_ref[...] + y_ref[...]) + b_ref[...]
    h = jnp.maximum(h, 0.0)
    j = pl.program_id(0)
    h_ref[...] = jnp.where(j == 0, h, h * h)


def _tc_combine(s1a, s1b, Y, hist_dst, b2d):
    nb = N // _TCB
    return pl.pallas_call(
        _tc_combine_body,
        grid=(2, nb),
        in_specs=[
            pl.BlockSpec((_TCB, D), lambda j, i: (i, 0)),
            pl.BlockSpec((_TCB, D), lambda j, i: (i, 0)),
            pl.BlockSpec((_TCB, D), lambda j, i: (i, 0)),
            pl.BlockSpec((_TCB, NW), lambda j, i: (i, 0)),
            pl.BlockSpec((1, D), lambda j, i: (0, 0)),
        ],
        out_specs=pl.BlockSpec((_TCB, D), lambda j, i: (j * nb + i, 0)),
        out_shape=jax.ShapeDtypeStruct((2 * N, D), _f32),
    )(s1a, s1b, Y, hist_dst, b2d)


def _tc_final_body(h_ref, a_ref, b_ref, hist_ref, g_ref):
    cnt = jnp.sum(hist_ref[...], axis=1)
    h = h_ref[...]
    sums = cnt[:, None] * h * h - 2.0 * h * a_ref[...] + b_ref[...]
    mean = sums / jnp.maximum(cnt, 1.0)[:, None]
    g_ref[...] = jnp.tanh(mean)


def _tc_final(hcat, A, B, hist_src):
    return pl.pallas_call(
        _tc_final_body,
        grid=(N // _TCB,),
        in_specs=[
            pl.BlockSpec((_TCB, D), lambda i: (i, 0)),
            pl.BlockSpec((_TCB, D), lambda i: (i, 0)),
            pl.BlockSpec((_TCB, D), lambda i: (i, 0)),
            pl.BlockSpec((_TCB, NW), lambda i: (i, 0)),
        ],
        out_specs=pl.BlockSpec((_TCB, D), lambda i: (i, 0)),
        out_shape=jax.ShapeDtypeStruct((N, D), _f32),
    )(hcat, A, B, hist_src)


@jax.jit
def kernel(X, edge_index, W, b):
    src = edge_index[0]
    dst = edge_index[1]

    srcR = src.reshape(NW, HNCH, HCH)
    dstR = dst.reshape(NW, HNCH, HCH)

    src_gs = src.reshape(NC, NS, NCH_GS, C)
    dst_gs = dst.reshape(NC, NS, NCH_GS, C)

    src_t = src.reshape(NS, NCH_AB, C)
    dst_t = dst.reshape(NS, NCH_AB, C)
    gidx_ab = jnp.stack([dst_t, dst_t + N])
    sidx_ab = jnp.stack([src_t, src_t])

    hists = _sc_hist(srcR, dstR)
    hist_src = hists[0].T
    hist_dst = hists[1].T

    Y = _tc_prep(X, W, hist_dst)
    S1 = _sc_gs(Y, src_gs, dst_gs)
    HCAT = _tc_combine(S1[0], S1[1], Y, hist_dst, b.reshape(1, D))
    AB = _sc_ab(HCAT, gidx_ab, sidx_ab)
    return _tc_final(HCAT[:N], AB[0], AB[1], hist_src)


# speedup vs baseline: 27.6374x; 1.0054x over previous
import functools

import jax
import jax.numpy as jnp
from jax import lax
from jax.experimental import pallas as pl
from jax.experimental.pallas import tpu as pltpu
from jax.experimental.pallas import tpu_sc as plsc

N = 10000
E = 320000
D = 128

NC = 2
NS = 16
NW = NC * NS
L = 16

HCH = 80
HNCH = (E // NW) // HCH

C = 125
NCH_GS = (E // NW) // C
NCH_AB = (E // NS) // C
RPT = N // NS

_mesh = plsc.VectorSubcoreMesh(core_axis_name="c", subcore_axis_name="s",
                               num_cores=NC, num_subcores=NS)
_f32 = jnp.float32
_sc_params = pltpu.CompilerParams(needs_layout_passes=False,
                                  use_tc_tiling_on_sc=False)


def _zero_ref(ref, rows):
    zeros = jnp.zeros((L,), _f32)

    def row(r, _):
        for j in range(D // L):
            ref[r, pl.ds(j * L, L)] = zeros
        return 0

    lax.fori_loop(0, rows, row, 0, unroll=2)


def _zero_acc_slice(buf, rows, acc):
    _zero_ref(buf, rows)
    base = lax.axis_index("s") * RPT
    for q in range(RPT // rows):
        pltpu.sync_copy(buf, acc.at[pl.ds(base + q * rows, rows)])
    rem = RPT % rows
    if rem:
        pltpu.sync_copy(buf.at[pl.ds(0, rem)],
                        acc.at[pl.ds(base + RPT - rem, rem)])


def _copy_out_slice(acc, out_hbm):
    c = lax.axis_index("c")
    base = lax.axis_index("s") * RPT
    pltpu.sync_copy(acc.at[pl.ds(base, RPT)], out_hbm.at[c, pl.ds(base, RPT)])


def _hist_body(src_hbm, dst_hbm, out_hbm, idx_s, idx_d, hist_s, hist_d):
    wid = lax.axis_index("s") * NC + lax.axis_index("c")
    pltpu.sync_copy(src_hbm.at[wid], idx_s)
    pltpu.sync_copy(dst_hbm.at[wid], idx_d)

    zeros = jnp.zeros((L,), _f32)

    def zb(i, _):
        hist_s[pl.ds(i * L, L)] = zeros
        hist_d[pl.ds(i * L, L)] = zeros
        return 0

    lax.fori_loop(0, N // L, zb, 0, unroll=4)

    ones = jnp.ones((L,), _f32)

    def hb(r, _):
        for j in range(HCH // L):
            vs = idx_s[r, pl.ds(j * L, L)]
            plsc.addupdate_scatter(hist_s, [vs], ones)
            vd = idx_d[r, pl.ds(j * L, L)]
            plsc.addupdate_scatter(hist_d, [vd], ones)
        return 0

    lax.fori_loop(0, HNCH, hb, 0)

    pltpu.sync_copy(hist_s, out_hbm.at[0, wid])
    pltpu.sync_copy(hist_d, out_hbm.at[1, wid])


_sc_hist = functools.partial(
    pl.kernel,
    out_type=jax.ShapeDtypeStruct((2, NW, N), _f32),
    mesh=_mesh,
    compiler_params=_sc_params,
    scratch_types=[
        pltpu.VMEM((HNCH, HCH), jnp.int32),
        pltpu.VMEM((HNCH, HCH), jnp.int32),
        pltpu.VMEM((N,), _f32),
        pltpu.VMEM((N,), _f32),
    ],
)(_hist_body)


def _gscat_pipeline(table_hbm, gidx_hbm, sidx_hbm, acc, gbufs, gsems,
                    ibuf_g, ibuf_s, isems, ssems, c, sid, nch):
    def fire_i(t, s):
        pltpu.async_copy(gidx_hbm.at[c, sid, t], ibuf_g.at[s], isems[s])
        pltpu.async_copy(sidx_hbm.at[c, sid, t], ibuf_s.at[s], isems[s])

    def wait_i(s):
        pltpu.make_async_copy(gidx_hbm.at[c, sid, 0], ibuf_g.at[s],
                              isems[s]).wait()
        pltpu.make_async_copy(gidx_hbm.at[c, sid, 0], ibuf_s.at[s],
                              isems[s]).wait()

    def fire_g(s, b):
        pltpu.async_copy(table_hbm.at[ibuf_g.at[s]], gbufs[b], gsems[b])

    def wait_g(b):
        pltpu.make_async_copy(table_hbm.at[ibuf_g.at[0]], gbufs[b],
                              gsems[b]).wait()

    def scat(s, b):
        pltpu.async_copy(gbufs[b], acc.at[ibuf_s.at[s]], ssems[b], add=True)

    def wait_s(b):
        pltpu.make_async_copy(gbufs[b], acc.at[ibuf_s.at[0]],
                              ssems[b]).wait()

    def sub(t, tm3, tm4, do_wait_s=True, do_fire_i=True, do_fire_g=True):
        if do_wait_s:
            wait_s((tm3 + 2) % 3)
        if do_fire_i:
            fire_i(t + 3, (tm4 + 3) % 4)
        if do_fire_g:
            wait_i((tm4 + 2) % 4)
            fire_g((tm4 + 2) % 4, (tm3 + 2) % 3)
        wait_g(tm3)
        scat(tm4, tm3)

    fire_i(0, 0)
    fire_i(1, 1)
    fire_i(2, 2)
    wait_i(0)
    fire_g(0, 0)
    wait_i(1)
    fire_g(1, 1)

    sub(0, 0, 0, do_wait_s=False)
    for t in range(1, 12):
        sub(t, t % 3, t % 4)

    main12 = (nch - 15) // 12

    def block(i, _):
        tb = 12 * i + 12
        for u in range(12):
            sub(tb + u, u % 3, u % 4)
        return 0

    lax.fori_loop(0, main12, block, 0)

    for t in range(12 + 12 * main12, nch):
        sub(t, t % 3, t % 4,
            do_fire_i=(t + 3 < nch), do_fire_g=(t + 2 < nch))

    wait_s((nch - 1) % 3)


def _make_gscat(nch):
    def body(table_hbm, gidx_hbm, sidx_hbm, out_hbm,
             gbuf0, gbuf1, gbuf2, ibuf_g, ibuf_s, acc,
             gsem0, gsem1, gsem2, isem0, isem1, isem2, isem3,
             ssem0, ssem1, ssem2):
        c = lax.axis_index("c")
        sid = lax.axis_index("s")
        _zero_acc_slice(gbuf0, C, acc)
        plsc.subcore_barrier()
        _gscat_pipeline(table_hbm, gidx_hbm, sidx_hbm, acc,
                        (gbuf0, gbuf1, gbuf2),
                        (gsem0, gsem1, gsem2),
                        ibuf_g, ibuf_s,
                        (isem0, isem1, isem2, isem3),
                        (ssem0, ssem1, ssem2),
                        c, sid, nch)
        plsc.subcore_barrier()
        _copy_out_slice(acc, out_hbm)

    return functools.partial(
        pl.kernel,
        out_type=jax.ShapeDtypeStruct((NC, N, D), _f32),
        mesh=_mesh,
        compiler_params=_sc_params,
        scratch_types=[
            pltpu.VMEM((C, D), _f32),
            pltpu.VMEM((C, D), _f32),
            pltpu.VMEM((C, D), _f32),
            pltpu.VMEM((4, C), jnp.int32),
            pltpu.VMEM((4, C), jnp.int32),
            pltpu.VMEM_SHARED((N, D), _f32),
            pltpu.SemaphoreType.DMA,
            pltpu.SemaphoreType.DMA,
            pltpu.SemaphoreType.DMA,
            pltpu.SemaphoreType.DMA,
            pltpu.SemaphoreType.DMA,
            pltpu.SemaphoreType.DMA,
            pltpu.SemaphoreType.DMA,
            pltpu.SemaphoreType.DMA,
            pltpu.SemaphoreType.DMA,
            pltpu.SemaphoreType.DMA,
        ],
    )(body)


_sc_gs = _make_gscat(NCH_GS)
_sc_ab = _make_gscat(NCH_AB)


_TCB = 2000


def _tc_prep_body(x_ref, w_ref, hist_ref, y_ref):
    deg = 1.0 + jnp.sum(hist_ref[...], axis=1)
    dinv = lax.rsqrt(deg)
    xw = jnp.dot(x_ref[...], w_ref[...], preferred_element_type=_f32)
    y_ref[...] = xw * dinv[:, None]


def _tc_prep(X, W, hist_dst):
    return pl.pallas_call(
        _tc_prep_body,
        grid=(N // _TCB,),
        in_specs=[
            pl.BlockSpec((_TCB, D), lambda i: (i, 0)),
            pl.BlockSpec((D, D), lambda i: (0, 0)),
            pl.BlockSpec((_TCB, NW), lambda i: (i, 0)),
        ],
        out_specs=pl.BlockSpec((_TCB, D), lambda i: (i, 0)),
        out_shape=jax.ShapeDtypeStruct((N, D), _f32),
    )(X, W, hist_dst)


def _tc_combine_body(s1a_ref, s1b_ref, y_ref, hist_ref, b_ref, h_ref):
    deg = 1.0 + jnp.sum(hist_ref[...], axis=1)
    dinv = lax.rsqrt(deg)
    h = dinv[:, None] * (s1a_ref[...] + s1b_ref[...] + y_ref[...]) + b_ref[...]
    h = jnp.maximum(h, 0.0)
    j = pl.program_id(0)
    h_ref[...] = jnp.where(j == 0, h, h * h)


def _tc_combine(s1a, s1b, Y, hist_dst, b2d):
    nb = N // _TCB
    return pl.pallas_call(
        _tc_combine_body,
        grid=(2, nb),
        in_specs=[
            pl.BlockSpec((_TCB, D), lambda j, i: (i, 0)),
            pl.BlockSpec((_TCB, D), lambda j, i: (i, 0)),
            pl.BlockSpec((_TCB, D), lambda j, i: (i, 0)),
            pl.BlockSpec((_TCB, NW), lambda j, i: (i, 0)),
            pl.BlockSpec((1, D), lambda j, i: (0, 0)),
        ],
        out_specs=pl.BlockSpec((_TCB, D), lambda j, i: (j * nb + i, 0)),
        out_shape=jax.ShapeDtypeStruct((2 * N, D), _f32),
    )(s1a, s1b, Y, hist_dst, b2d)


def _tc_final_body(h_ref, a_ref, b_ref, hist_ref, g_ref):
    cnt = jnp.sum(hist_ref[...], axis=1)
    h = h_ref[...]
    sums = cnt[:, None] * h * h - 2.0 * h * a_ref[...] + b_ref[...]
    mean = sums / jnp.maximum(cnt, 1.0)[:, None]
    g_ref[...] = jnp.tanh(mean)


def _tc_final(hcat, A, B, hist_src):
    return pl.pallas_call(
        _tc_final_body,
        grid=(N // _TCB,),
        in_specs=[
            pl.BlockSpec((_TCB, D), lambda i: (i, 0)),
            pl.BlockSpec((_TCB, D), lambda i: (i, 0)),
            pl.BlockSpec((_TCB, D), lambda i: (i, 0)),
            pl.BlockSpec((_TCB, NW), lambda i: (i, 0)),
        ],
        out_specs=pl.BlockSpec((_TCB, D), lambda i: (i, 0)),
        out_shape=jax.ShapeDtypeStruct((N, D), _f32),
    )(hcat, A, B, hist_src)


@jax.jit
def kernel(X, edge_index, W, b):
    src = edge_index[0]
    dst = edge_index[1]

    srcR = src.reshape(NW, HNCH, HCH)
    dstR = dst.reshape(NW, HNCH, HCH)

    src_gs = src.reshape(NC, NS, NCH_GS, C)
    dst_gs = dst.reshape(NC, NS, NCH_GS, C)

    src_t = src.reshape(NS, NCH_AB, C)
    dst_t = dst.reshape(NS, NCH_AB, C)
    gidx_ab = jnp.stack([dst_t, dst_t + N])
    sidx_ab = jnp.stack([src_t, src_t])

    hists = _sc_hist(srcR, dstR)
    hist_src = hists[0].T
    hist_dst = hists[1].T

    Y = _tc_prep(X, W, hist_dst)
    S1 = _sc_gs(Y, src_gs, dst_gs)
    HCAT = _tc_combine(S1[0], S1[1], Y, hist_dst, b.reshape(1, D))
    AB = _sc_ab(HCAT, gidx_ab, sidx_ab)
    return _tc_final(HCAT, AB[0], AB[1], hist_src)
